# baseline (speedup 1.0000x reference)
import functools
import math

import jax
import jax.numpy as jnp
from jax import lax
from jax.experimental import pallas as pl
from jax.experimental.pallas import tpu as pltpu
from jax.experimental.pallas import tpu_sc as plsc

B = 4096
L = 50
D = 128
NC = 2
NS = 16
NW = NC * NS
BPW = B // NW
RG = 4
NIDX = RG * L
NSTEP = BPW // RG
NBUF = 4
NACC = 5
SCALE = 1.0 / math.sqrt(float(L))

_mesh = plsc.VectorSubcoreMesh(core_axis_name="c", subcore_axis_name="s")


@functools.partial(
    pl.kernel,
    mesh=_mesh,
    out_type=jax.ShapeDtypeStruct((B, D), jnp.float32),
    scratch_types=[
        pltpu.VMEM((BPW * L,), jnp.int32),
        pltpu.VMEM((NBUF, NIDX, D), jnp.float32),
        pltpu.VMEM((BPW, D), jnp.float32),
        pltpu.SemaphoreType.DMA,
        pltpu.SemaphoreType.DMA,
        pltpu.SemaphoreType.DMA,
        pltpu.SemaphoreType.DMA,
    ],
)
def _embed(table_hbm, idx_hbm, out_hbm, idx_v, rows_v, out_v, s0, s1, s2, s3):
    c = lax.axis_index("c")
    s = lax.axis_index("s")
    wid = s * NC + c
    base = wid * (BPW * L)
    pltpu.sync_copy(idx_hbm.at[pl.ds(base, BPW * L)], idx_v)
    sems = [s0, s1, s2, s3]

    splits = ((0, 56), (56, 48), (104, 48), (152, 48))

    def gather_group(t, buf, sem):
        i0 = t * NIDX
        return [pltpu.make_async_copy(
                    table_hbm.at[idx_v.at[pl.ds(i0 + off, n)]],
                    rows_v.at[buf].at[pl.ds(off, n)], sem)
                for off, n in splits]

    def fire(t, buf, sem):
        for d in gather_group(t, buf, sem):
            d.start()

    for p in range(NBUF - 1):
        fire(p, p, sems[p])

    def step(i, carry):
        for b in range(NBUF):
            t = i * NBUF + b
            for d in gather_group(t, b, sems[b]):
                d.wait()
            tn = t + NBUF - 1
            nb = (NBUF - 1 + b) % NBUF

            @pl.when(tn < NSTEP)
            def _():
                fire(tn, nb, sems[nb])

            sls = [pl.ds(ch * 16, 16) for ch in range(D // 16)]
            for rp in range(RG // 2):
                r0 = rp * 2
                bases = [r0 * L, (r0 + 1) * L]
                init = tuple(rows_v[b, j0, sl] for j0 in bases for sl in sls)

                def jbody(j, accs, _b=b, _bases=bases):
                    new = []
                    for p in range(2):
                        for ch in range(D // 16):
                            new.append(accs[p * (D // 16) + ch]
                                       + rows_v[_b, _bases[p] + j, sls[ch]])
                    return tuple(new)

                accs = lax.fori_loop(1, L, jbody, init)
                for p in range(2):
                    row = t * RG + r0 + p
                    for ch in range(D // 16):
                        out_v[row, sls[ch]] = accs[p * (D // 16) + ch] * SCALE
        return carry

    lax.fori_loop(0, NSTEP // NBUF, step, 0)
    pltpu.sync_copy(out_v, out_hbm.at[pl.ds(wid * BPW, BPW)])


def kernel(inputs, table):
    idx = inputs.astype(jnp.int32).reshape(-1)
    return _embed(table, idx)
